# speedup vs baseline: 4.5218x; 4.5218x over previous
import functools

import jax
import jax.numpy as jnp
from jax import lax
from jax.experimental import pallas as pl
from jax.experimental.pallas import tpu as pltpu
from jax.experimental.pallas import tpu_sc as plsc

D_MODEL = 128

_NC = 2
_NS = 16
_NW = _NC * _NS

_G = 128
_KG = 2
_C = _G * _KG


def _gather_add(xf, idx2, table):
    B = xf.shape[0]
    b_per_w = B // _NW
    n_chunks = b_per_w // _C
    g_per_w = b_per_w // _G

    mesh = plsc.VectorSubcoreMesh(
        core_axis_name="c", subcore_axis_name="s",
        num_cores=_NC, num_subcores=_NS)

    @functools.partial(
        pl.kernel,
        mesh=mesh,
        out_type=jax.ShapeDtypeStruct((B, D_MODEL), jnp.float32),
        scratch_types=[
            pltpu.VMEM((_KG, _G), jnp.int32),
            pltpu.VMEM((_C, D_MODEL), jnp.float32),
            pltpu.SemaphoreType.DMA,
            pltpu.SemaphoreType.DMA,
        ],
    )
    def k(x_hbm, idx_hbm, table_hbm, out_hbm, idx_v, acc_v, sem_x, sem_g):
        wid = lax.axis_index("s") * _NC + lax.axis_index("c")
        row0 = wid * b_per_w
        grp0 = wid * g_per_w

        def step(i, carry):
            off = row0 + i * _C
            goff = grp0 + i * _KG
            pltpu.sync_copy(idx_hbm.at[pl.ds(goff, _KG)], idx_v)
            pltpu.async_copy(x_hbm.at[pl.ds(off, _C)], acc_v, sem_x).wait()
            cps = [
                pltpu.async_copy(
                    table_hbm.at[idx_v.at[j]],
                    acc_v.at[pl.ds(j * _G, _G)],
                    sem_g,
                    add=True,
                )
                for j in range(_KG)
            ]
            for cp in cps:
                cp.wait()
            pltpu.sync_copy(acc_v, out_hbm.at[pl.ds(off, _C)])
            return carry

        lax.fori_loop(0, n_chunks, step, 0)

    return k(xf, idx2, table)


def kernel(x, time, embeddings):
    bt, s, d = x.shape
    b = bt * s
    xf = x.reshape(b, d)
    idx2 = time.reshape(b // _G, _G).astype(jnp.int32)
    out = _gather_add(xf, idx2, embeddings)
    return out.reshape(bt, s, d)


# speedup vs baseline: 5.8246x; 1.2881x over previous
import functools

import jax
import jax.numpy as jnp
from jax import lax
from jax.experimental import pallas as pl
from jax.experimental.pallas import tpu as pltpu
from jax.experimental.pallas import tpu_sc as plsc

D_MODEL = 128

_NC = 2
_NS = 16
_NW = _NC * _NS

_G = 128
_C = _G
_NBUF = 5
_PF = 3


def _gather_add(xf, idx2, table):
    B = xf.shape[0]
    b_per_w = B // _NW
    n = b_per_w // _C
    assert n % _NBUF == 0

    mesh = plsc.VectorSubcoreMesh(
        core_axis_name="c", subcore_axis_name="s",
        num_cores=_NC, num_subcores=_NS)

    @functools.partial(
        pl.kernel,
        mesh=mesh,
        out_type=jax.ShapeDtypeStruct((B, D_MODEL), jnp.float32),
        scratch_types=[
            pltpu.VMEM((n, _G), jnp.int32),
            pltpu.VMEM((_NBUF, _C, D_MODEL), jnp.float32),
            [pltpu.SemaphoreType.DMA] * _NBUF,
            [pltpu.SemaphoreType.DMA] * _NBUF,
            [pltpu.SemaphoreType.DMA] * _NBUF,
        ],
    )
    def k(x_hbm, idx_hbm, table_hbm, out_hbm, idx_v, acc_v,
          sem_x, sem_g, sem_o):
        wid = lax.axis_index("s") * _NC + lax.axis_index("c")
        row0 = wid * b_per_w

        pltpu.sync_copy(idx_hbm.at[wid], idx_v)

        def start_x(c, slot):
            pltpu.async_copy(
                x_hbm.at[pl.ds(row0 + c * _C, _C)], acc_v.at[slot],
                sem_x[slot])

        def wait_x(c, slot):
            pltpu.make_async_copy(
                x_hbm.at[pl.ds(row0 + c * _C, _C)], acc_v.at[slot],
                sem_x[slot]).wait()

        def wait_out(c, slot):
            pltpu.make_async_copy(
                acc_v.at[slot], out_hbm.at[pl.ds(row0 + c * _C, _C)],
                sem_o[slot]).wait()

        for b in range(_PF):
            start_x(b, b)

        def outer(j, carry):
            for b in range(_NBUF):
                c = j * _NBUF + b

                @pl.when(c < n - _PF)
                def _():
                    slot_n = (b + _PF) % _NBUF

                    def drain_and_fetch():
                        wait_out(c - (_NBUF - _PF), slot_n)
                        start_x(c + _PF, slot_n)

                    if b < _NBUF - _PF:
                        @pl.when(j >= 1)
                        def _():
                            drain_and_fetch()

                        @pl.when(j < 1)
                        def _():
                            start_x(c + _PF, slot_n)
                    else:
                        drain_and_fetch()

                wait_x(c, b)
                pltpu.async_copy(
                    table_hbm.at[idx_v.at[c]], acc_v.at[b],
                    sem_g[b], add=True).wait()
                pltpu.async_copy(
                    acc_v.at[b], out_hbm.at[pl.ds(row0 + c * _C, _C)],
                    sem_o[b])
            return carry

        lax.fori_loop(0, n // _NBUF, outer, 0)

        for i in range(_NBUF):
            c = n - _NBUF + i
            wait_out(c, c % _NBUF)

    return k(xf, idx2, table)


def kernel(x, time, embeddings):
    bt, s, d = x.shape
    b = bt * s
    xf = x.reshape(b, d)
    idx2 = time.reshape(_NW, b // (_NW * _G), _G).astype(jnp.int32)
    out = _gather_add(xf, idx2, embeddings)
    return out.reshape(bt, s, d)


# speedup vs baseline: 6.0789x; 1.0437x over previous
import functools

import jax
import jax.numpy as jnp
from jax import lax
from jax.experimental import pallas as pl
from jax.experimental.pallas import tpu as pltpu
from jax.experimental.pallas import tpu_sc as plsc

D_MODEL = 128

_NC = 2
_NS = 16
_NW = _NC * _NS

_G = 128
_C = _G
_NBUF = 5
_PF = 3


def _gather_add(xf, idx2, table):
    B = xf.shape[0]
    b_per_w = B // _NW
    n = b_per_w // _C
    assert n % _NBUF == 0

    mesh = plsc.VectorSubcoreMesh(
        core_axis_name="c", subcore_axis_name="s",
        num_cores=_NC, num_subcores=_NS)

    @functools.partial(
        pl.kernel,
        mesh=mesh,
        out_type=jax.ShapeDtypeStruct((B, D_MODEL), jnp.float32),
        scratch_types=[
            pltpu.VMEM((n, _G), jnp.int32),
            pltpu.VMEM((_NBUF, _C, D_MODEL), jnp.float32),
            [pltpu.SemaphoreType.DMA] * _NBUF,
            [pltpu.SemaphoreType.DMA] * _NBUF,
            [pltpu.SemaphoreType.DMA] * _NBUF,
        ],
    )
    def k(x_hbm, idx_hbm, table_hbm, out_hbm, idx_v, acc_v,
          sem_x, sem_g, sem_o):
        wid = lax.axis_index("s") * _NC + lax.axis_index("c")
        row0 = wid * b_per_w

        pltpu.sync_copy(idx_hbm.at[wid], idx_v)

        def start_x(c, slot):
            pltpu.async_copy(
                x_hbm.at[pl.ds(row0 + c * _C, _C)], acc_v.at[slot],
                sem_x[slot])

        def wait_x(c, slot):
            pltpu.make_async_copy(
                x_hbm.at[pl.ds(row0 + c * _C, _C)], acc_v.at[slot],
                sem_x[slot]).wait()

        def wait_out(c, slot):
            pltpu.make_async_copy(
                acc_v.at[slot], out_hbm.at[pl.ds(row0 + c * _C, _C)],
                sem_o[slot]).wait()

        def start_gather(c, slot):
            pltpu.async_copy(
                table_hbm.at[idx_v.at[c]], acc_v.at[slot],
                sem_g[slot], add=True)

        def wait_gather(c, slot):
            pltpu.make_async_copy(
                table_hbm.at[idx_v.at[c]], acc_v.at[slot],
                sem_g[slot]).wait()

        for b in range(_PF):
            start_x(b, b)
        wait_x(0, 0)
        start_gather(0, 0)

        def outer(j, carry):
            for b in range(_NBUF):
                c = j * _NBUF + b

                @pl.when(c < n - _PF)
                def _():
                    slot_n = (b + _PF) % _NBUF

                    def drain_and_fetch():
                        wait_out(c - (_NBUF - _PF), slot_n)
                        start_x(c + _PF, slot_n)

                    if b < _NBUF - _PF:
                        @pl.when(j >= 1)
                        def _():
                            drain_and_fetch()

                        @pl.when(j < 1)
                        def _():
                            start_x(c + _PF, slot_n)
                    else:
                        drain_and_fetch()

                @pl.when(c < n - 1)
                def _():
                    wait_x(c + 1, (b + 1) % _NBUF)
                    start_gather(c + 1, (b + 1) % _NBUF)

                wait_gather(c, b)
                pltpu.async_copy(
                    acc_v.at[b], out_hbm.at[pl.ds(row0 + c * _C, _C)],
                    sem_o[b])
            return carry

        lax.fori_loop(0, n // _NBUF, outer, 0)

        for i in range(_NBUF):
            c = n - _NBUF + i
            wait_out(c, c % _NBUF)

    return k(xf, idx2, table)


def kernel(x, time, embeddings):
    bt, s, d = x.shape
    b = bt * s
    xf = x.reshape(b, d)
    idx2 = time.reshape(_NW, b // (_NW * _G), _G).astype(jnp.int32)
    out = _gather_add(xf, idx2, embeddings)
    return out.reshape(bt, s, d)
